# Initial kernel scaffold; baseline (speedup 1.0000x reference)
import jax
import jax.numpy as jnp
from jax.experimental import pallas as pl


def kernel(bincounts, genes_oi, labels, local_cellxgene_ix, binixs, baseline_weight, differential_weight, cluster_modifier):
    raise NotImplementedError("write your pallas kernel here")


# speedup vs baseline: 1.0186x; 1.0186x over previous
import functools

import jax
import jax.numpy as jnp
from jax.experimental import pallas as pl
from jax.experimental.pallas import tpu as pltpu

C = 512
G = 100
F = 320
K = 16
CB = 8


def _ln_factorial(n_f32):
    x = jnp.maximum(n_f32, 2.0)
    inv = 1.0 / x
    inv2 = inv * inv
    series = inv * (1.0 / 12.0 + inv2 * (-1.0 / 360.0 + inv2 * (1.0 / 1260.0)))
    half_ln_2pi = 0.9189385332046727
    stir = (x + 0.5) * jnp.log(x) - x + half_ln_2pi + series
    return jnp.where(n_f32 < 1.5, 0.0, stir)


def _etable_body(b_ref, h_ref, a_ref, m_ref, e_ref):
    u = (a_ref[...][:, :, None] * b_ref[...][None, :, :]
         + h_ref[...][None, :, :] + m_ref[...][:, :, None])
    e_ref[...] = jnp.sum(jnp.exp(u), axis=-1)


def _reduce_body(cnt_ref, b_ref, h_ref, a_ref, m_ref, e_ref, out_ref):
    cnt = cnt_ref[...].astype(jnp.float32)
    u = (a_ref[...][:, :, None] * b_ref[...][None, :, :]
         + h_ref[...][None, :, :] + m_ref[...][:, :, None])
    t = cnt * u - _ln_factorial(cnt)
    out_ref[...] = jnp.sum(t, axis=-1) - e_ref[...]


def kernel(bincounts, genes_oi, labels, local_cellxgene_ix, binixs,
           baseline_weight, differential_weight, cluster_modifier):
    b = bincounts.astype(jnp.float32)
    h = jnp.take(baseline_weight, genes_oi, axis=0)
    a_k = differential_weight.reshape(K, 1)
    m_k = cluster_modifier.reshape(K, 1)

    e_tab = pl.pallas_call(
        _etable_body,
        out_shape=jax.ShapeDtypeStruct((K, G), jnp.float32),
    )(b, h, a_k, m_k)

    a_c = jnp.take(a_k[:, 0], labels)[:, None]
    m_c = jnp.take(m_k[:, 0], labels)[:, None]
    e_c = jnp.take(e_tab, labels, axis=0)

    idx = local_cellxgene_ix * F + binixs
    count = jnp.zeros((C * G * F,), jnp.int32).at[idx].add(1).reshape(C, G, F)

    out = pl.pallas_call(
        _reduce_body,
        grid=(C // CB,),
        in_specs=[
            pl.BlockSpec((CB, G, F), lambda i: (i, 0, 0)),
            pl.BlockSpec((G, F), lambda i: (0, 0)),
            pl.BlockSpec((G, F), lambda i: (0, 0)),
            pl.BlockSpec((CB, 1), lambda i: (i, 0)),
            pl.BlockSpec((CB, 1), lambda i: (i, 0)),
            pl.BlockSpec((CB, G), lambda i: (i, 0)),
        ],
        out_specs=pl.BlockSpec((CB, G), lambda i: (i, 0)),
        out_shape=jax.ShapeDtypeStruct((C, G), jnp.float32),
    )(count, b, h, a_c, m_c, e_c)
    return out


# speedup vs baseline: 10.0923x; 9.9081x over previous
import functools

import jax
import jax.numpy as jnp
from jax import lax
from jax.experimental import pallas as pl
from jax.experimental.pallas import tpu as pltpu
from jax.experimental.pallas import tpu_sc as plsc

C = 512
G = 100
F = 320
K = 16
CB = 8
NFRAG = 8_000_000
BINS = C * G * F

NT = 32
FPT = NFRAG // NT
SLAB_BITS = 17
NSLAB = BINS >> SLAB_BITS
WIN = 1 << 16
NWIN = BINS // WIN
QUANT = 256
RING = 512
PART = NFRAG + NT * NSLAB * QUANT
SENT = 1 << 30

CHW = 4096
NCH = -(-FPT // CHW)
P2CHW = 2048

_SC_MESH = plsc.VectorSubcoreMesh(core_axis_name="sc_core",
                                  subcore_axis_name="sc_tile")
_SC_PARAMS = pltpu.CompilerParams(needs_layout_passes=False)

_LANE = lambda: lax.broadcasted_iota(jnp.int32, (16,), 0)


def _extract(vec16, lane):
    return jnp.sum(jnp.where(_LANE() == lane, vec16, 0))


def _gather_scalar(ref, idx_scalar):
    g = plsc.load_gather(ref, [jnp.full((16,), idx_scalar, jnp.int32)])
    return jnp.max(g)


def _pack_body(cxg_ref, bin_ref, out_ref):
    out_ref[...] = cxg_ref[...] * F + bin_ref[...]


def _pack(cxg, bins):
    rows, cols = 1000, 8000
    out = pl.pallas_call(
        _pack_body,
        grid=(125,),
        in_specs=[pl.BlockSpec((8, cols), lambda i: (i, 0)),
                  pl.BlockSpec((8, cols), lambda i: (i, 0))],
        out_specs=pl.BlockSpec((8, cols), lambda i: (i, 0)),
        out_shape=jax.ShapeDtypeStruct((rows, cols), jnp.int32),
    )(cxg.reshape(rows, cols), bins.reshape(rows, cols))
    return out.reshape(NFRAG)


@functools.partial(
    pl.kernel, mesh=_SC_MESH, compiler_params=_SC_PARAMS,
    out_type=jax.ShapeDtypeStruct((NT * 128,), jnp.int32),
    scratch_types=[pltpu.VMEM((128,), jnp.int32),
                   pltpu.VMEM((CHW,), jnp.int32)],
)
def _sc_count(idx_hbm, out_hbm, cnt_v, chunk_v):
    tid = lax.axis_index("sc_tile") * 2 + lax.axis_index("sc_core")
    base = tid * FPT
    for k in range(8):
        cnt_v[pl.ds(k * 16, 16)] = jnp.zeros((16,), jnp.int32)

    def chunk_body(i, _):
        rs = jnp.minimum(i * CHW, FPT - CHW)
        src = pl.multiple_of(base + rs, 16)
        pltpu.sync_copy(idx_hbm.at[pl.ds(src, CHW)], chunk_v)
        lo = i * CHW

        def vec_body(j, _):
            v = chunk_v[pl.ds(j * 16, 16)]
            pos = rs + j * 16 + _LANE()
            m = pos >= lo
            s = lax.shift_right_logical(v, SLAB_BITS)
            r, lastm = plsc.scan_count(s, mask=m)
            plsc.addupdate_scatter(cnt_v, [jnp.where(m, s, 0)], r, mask=lastm)
            return 0

        lax.fori_loop(0, CHW // 16, vec_body, 0)
        return 0

    lax.fori_loop(0, NCH, chunk_body, 0)
    pltpu.sync_copy(cnt_v, out_hbm.at[pl.ds(pl.multiple_of(tid * 128, 128), 128)])


@functools.partial(
    pl.kernel, mesh=_SC_MESH, compiler_params=_SC_PARAMS,
    out_type=jax.ShapeDtypeStruct((PART,), jnp.int32),
    scratch_types=[pltpu.VMEM((128,), jnp.int32),
                   pltpu.VMEM((128,), jnp.int32),
                   pltpu.VMEM((128,), jnp.int32),
                   pltpu.VMEM((NSLAB * RING,), jnp.int32),
                   pltpu.VMEM((CHW,), jnp.int32)],
)
def _sc_partition(idx_hbm, base_hbm, part_hbm, hbase_v, fill_v, flq_v,
                  rings_v, chunk_v):
    tid = lax.axis_index("sc_tile") * 2 + lax.axis_index("sc_core")
    base = tid * FPT
    pltpu.sync_copy(base_hbm.at[pl.ds(pl.multiple_of(tid * 128, 128), 128)],
                    hbase_v)
    for k in range(8):
        fill_v[pl.ds(k * 16, 16)] = jnp.zeros((16,), jnp.int32)
        flq_v[pl.ds(k * 16, 16)] = jnp.zeros((16,), jnp.int32)

    def flush_block(blk):
        def pending_count():
            fi = fill_v[pl.ds(blk * 16, 16)]
            qi = flq_v[pl.ds(blk * 16, 16)]
            return jnp.sum(jnp.where(fi - qi >= QUANT, 1, 0))

        def cond(n):
            return n > 0

        def body(n):
            fi = fill_v[pl.ds(blk * 16, 16)]
            qi = flq_v[pl.ds(blk * 16, 16)]
            m = fi - qi >= QUANT
            lane = jnp.max(plsc.all_reduce_ffs(m))
            s = blk * 16 + lane
            q = _extract(qi, lane)
            b = _extract(hbase_v[pl.ds(blk * 16, 16)], lane)
            ringoff = pl.multiple_of(s * RING + (q & (RING - 1)), QUANT)
            pltpu.sync_copy(rings_v.at[pl.ds(ringoff, QUANT)],
                            part_hbm.at[pl.ds(pl.multiple_of(b + q, QUANT),
                                              QUANT)])
            flq_v[pl.ds(blk * 16, 16)] = qi + jnp.where(_LANE() == lane, QUANT, 0)
            return n - 1

        lax.while_loop(cond, body, pending_count())

    def chunk_body(i, _):
        rs = jnp.minimum(i * CHW, FPT - CHW)
        src = pl.multiple_of(base + rs, 16)
        pltpu.sync_copy(idx_hbm.at[pl.ds(src, CHW)], chunk_v)
        lo = i * CHW

        def group_body(g, _):
            for jj in range(16):
                j = g * 16 + jj
                v = chunk_v[pl.ds(j * 16, 16)]
                pos = rs + j * 16 + _LANE()
                m = pos >= lo
                s = lax.shift_right_logical(v, SLAB_BITS)
                s = jnp.where(m, s, 0)
                r, lastm = plsc.scan_count(s, mask=m)
                f = plsc.load_gather(fill_v, [s])
                slot = (f + r - 1) & (RING - 1)
                plsc.store_scatter(rings_v, [s * RING + slot], v, mask=m)
                plsc.addupdate_scatter(fill_v, [s], r, mask=lastm)
            for blk in range(8):
                flush_block(blk)
            return 0

        lax.fori_loop(0, CHW // 256, group_body, 0)
        return 0

    lax.fori_loop(0, NCH, chunk_body, 0)

    def drain_body(s, _):
        f = _gather_scalar(fill_v, s)
        q = _gather_scalar(flq_v, s)
        pend = f - q

        @pl.when(pend > 0)
        def _():
            end = q + QUANT
            for it in range(QUANT // 16):
                p = f + it * 16 + _LANE()
                m = p < end
                slot = p & (RING - 1)
                plsc.store_scatter(rings_v, [s * RING + slot],
                                   jnp.full((16,), SENT, jnp.int32), mask=m)
            b = _gather_scalar(hbase_v, s)
            ringoff = pl.multiple_of(s * RING + (q & (RING - 1)), QUANT)
            pltpu.sync_copy(rings_v.at[pl.ds(ringoff, QUANT)],
                            part_hbm.at[pl.ds(pl.multiple_of(b + q, QUANT),
                                              QUANT)])
        return 0

    lax.fori_loop(0, NSLAB, drain_body, 0)


@functools.partial(
    pl.kernel, mesh=_SC_MESH, compiler_params=_SC_PARAMS,
    out_type=jax.ShapeDtypeStruct((BINS,), jnp.int32),
    scratch_types=[pltpu.VMEM((256,), jnp.int32),
                   pltpu.VMEM((WIN,), jnp.int32),
                   pltpu.VMEM((P2CHW,), jnp.int32)],
)
def _sc_hist(part_hbm, bounds_hbm, hist_hbm, bounds_v, hist_v, chunk_v):
    tid = lax.axis_index("sc_tile") * 2 + lax.axis_index("sc_core")
    pltpu.sync_copy(bounds_hbm, bounds_v)

    for i in range(8):
        w = tid + i * NT

        @pl.when(w < NWIN)
        def _():
            s = lax.shift_right_logical(w, 1)
            sstart = _gather_scalar(bounds_v, s)
            send = _gather_scalar(bounds_v, 128 + s)
            wbase = w * WIN

            def zero_body(z, _):
                for k in range(8):
                    hist_v[pl.ds((z * 8 + k) * 16, 16)] = jnp.zeros((16,), jnp.int32)
                return 0

            lax.fori_loop(0, WIN // 128, zero_body, 0)

            n = send - sstart
            trips = lax.shift_right_logical(n + P2CHW - 1, 11)

            def trip_body(t, _):
                lo = sstart + t * P2CHW
                hi = jnp.minimum(lo + P2CHW, send)
                rs = pl.multiple_of(
                    jnp.minimum(lo, jnp.maximum(send - P2CHW, 0)), QUANT)
                pltpu.sync_copy(part_hbm.at[pl.ds(rs, P2CHW)], chunk_v)

                def vec_body(j, _):
                    v = chunk_v[pl.ds(j * 16, 16)]
                    pos = rs + j * 16 + _LANE()
                    lb = v - wbase
                    m = (pos >= lo) & (pos < hi) & (lb >= 0) & (lb < WIN)
                    lbs = jnp.where(m, lb, 0)
                    r, lastm = plsc.scan_count(lbs, mask=m)
                    plsc.addupdate_scatter(hist_v, [lbs], r, mask=lastm)
                    return 0

                lax.fori_loop(0, P2CHW // 16, vec_body, 0)
                return 0

            lax.fori_loop(0, trips, trip_body, 0)
            pltpu.sync_copy(hist_v,
                            hist_hbm.at[pl.ds(pl.multiple_of(w * WIN, WIN),
                                              WIN)])


def _ln_factorial(n_f32):
    x = jnp.maximum(n_f32, 2.0)
    inv = 1.0 / x
    inv2 = inv * inv
    series = inv * (1.0 / 12.0 + inv2 * (-1.0 / 360.0 + inv2 * (1.0 / 1260.0)))
    half_ln_2pi = 0.9189385332046727
    stir = (x + 0.5) * jnp.log(x) - x + half_ln_2pi + series
    return jnp.where(n_f32 < 1.5, 0.0, stir)


def _etable_body(b_ref, h_ref, a_ref, m_ref, e_ref):
    u = (a_ref[...][:, :, None] * b_ref[...][None, :, :]
         + h_ref[...][None, :, :] + m_ref[...][:, :, None])
    e_ref[...] = jnp.sum(jnp.exp(u), axis=-1)


def _reduce_body(cnt_ref, b_ref, h_ref, a_ref, m_ref, e_ref, out_ref):
    cnt = cnt_ref[...].astype(jnp.float32)
    u = (a_ref[...][:, :, None] * b_ref[...][None, :, :]
         + h_ref[...][None, :, :] + m_ref[...][:, :, None])
    t = cnt * u - _ln_factorial(cnt)
    out_ref[...] = jnp.sum(t, axis=-1) - e_ref[...]


def kernel(bincounts, genes_oi, labels, local_cellxgene_ix, binixs,
           baseline_weight, differential_weight, cluster_modifier):
    b = bincounts.astype(jnp.float32)
    h = jnp.take(baseline_weight, genes_oi, axis=0)
    a_k = differential_weight.reshape(K, 1)
    m_k = cluster_modifier.reshape(K, 1)

    e_tab = pl.pallas_call(
        _etable_body,
        out_shape=jax.ShapeDtypeStruct((K, G), jnp.float32),
    )(b, h, a_k, m_k)

    a_c = jnp.take(a_k[:, 0], labels)[:, None]
    m_c = jnp.take(m_k[:, 0], labels)[:, None]
    e_c = jnp.take(e_tab, labels, axis=0)

    idx = _pack(local_cellxgene_ix, binixs)
    counts = _sc_count(idx)

    cnt = counts.reshape(NT, 128)[:, :NSLAB]
    q = ((cnt + (QUANT - 1)) // QUANT) * QUANT
    flat = q.T.reshape(-1)
    starts = jnp.cumsum(flat) - flat
    base_ts = starts.reshape(NSLAB, NT).T
    base_in = jnp.zeros((NT, 128), jnp.int32).at[:, :NSLAB].set(base_ts)
    slab_tot = jnp.sum(q, axis=0)
    slab_start = jnp.cumsum(slab_tot) - slab_tot
    bounds = jnp.zeros((256,), jnp.int32)
    bounds = bounds.at[:NSLAB].set(slab_start)
    bounds = bounds.at[128:128 + NSLAB].set(slab_start + slab_tot)

    part = _sc_partition(idx, base_in.reshape(-1))
    hist = _sc_hist(part, bounds)

    out = pl.pallas_call(
        _reduce_body,
        grid=(C // CB,),
        in_specs=[
            pl.BlockSpec((CB, G, F), lambda i: (i, 0, 0)),
            pl.BlockSpec((G, F), lambda i: (0, 0)),
            pl.BlockSpec((G, F), lambda i: (0, 0)),
            pl.BlockSpec((CB, 1), lambda i: (i, 0)),
            pl.BlockSpec((CB, 1), lambda i: (i, 0)),
            pl.BlockSpec((CB, G), lambda i: (i, 0)),
        ],
        out_specs=pl.BlockSpec((CB, G), lambda i: (i, 0)),
        out_shape=jax.ShapeDtypeStruct((C, G), jnp.float32),
    )(hist.reshape(C, G, F), b, h, a_c, m_c, e_c)
    return out


# speedup vs baseline: 14.2055x; 1.4076x over previous
import functools

import jax
import jax.numpy as jnp
from jax import lax
from jax.experimental import pallas as pl
from jax.experimental.pallas import tpu as pltpu
from jax.experimental.pallas import tpu_sc as plsc

C = 512
G = 100
F = 320
K = 16
CB = 8
NFRAG = 8_000_000
BINS = C * G * F

NT = 32
FPT = NFRAG // NT
SLAB_BITS = 17
NSLAB = BINS >> SLAB_BITS
WIN = 1 << 16
NWIN = BINS // WIN
QUANT = 256
RING = 512
PART = NFRAG + NT * NSLAB * QUANT
SENT = 1 << 30

CHW = 16384
NCH = -(-FPT // CHW)
P2CHW = 8192
P2BITS = 13

_SC_MESH = plsc.VectorSubcoreMesh(core_axis_name="sc_core",
                                  subcore_axis_name="sc_tile")
_SC_PARAMS = pltpu.CompilerParams(needs_layout_passes=False)

_LANE = lambda: lax.broadcasted_iota(jnp.int32, (16,), 0)


def _extract(vec16, lane):
    return jnp.sum(jnp.where(_LANE() == lane, vec16, 0))


def _gather_scalar(ref, idx_scalar):
    g = plsc.load_gather(ref, [jnp.full((16,), idx_scalar, jnp.int32)])
    return jnp.max(g)


def _pack_body(cxg_ref, bin_ref, out_ref):
    out_ref[...] = cxg_ref[...] * F + bin_ref[...]


def _pack(cxg, bins):
    rows, cols = 1000, 8000
    out = pl.pallas_call(
        _pack_body,
        grid=(125,),
        in_specs=[pl.BlockSpec((8, cols), lambda i: (i, 0)),
                  pl.BlockSpec((8, cols), lambda i: (i, 0))],
        out_specs=pl.BlockSpec((8, cols), lambda i: (i, 0)),
        out_shape=jax.ShapeDtypeStruct((rows, cols), jnp.int32),
    )(cxg.reshape(rows, cols), bins.reshape(rows, cols))
    return out.reshape(NFRAG)


def _chunk_src(i, base):
    rs = jnp.minimum(i * CHW, FPT - CHW)
    return rs, pl.multiple_of(base + rs, 16)


def _dbuf_wait_issue(i, nch, idx_hbm, base, chunk_v, sem0, sem1):
    def _wait(sem, slot):
        _, src = _chunk_src(i, base)
        pltpu.make_async_copy(idx_hbm.at[pl.ds(src, CHW)],
                              chunk_v.at[pl.ds(slot * CHW, CHW)], sem).wait()

    def _issue(sem, slot):
        @pl.when(i + 1 < nch)
        def _():
            _, src = _chunk_src(i + 1, base)
            pltpu.async_copy(idx_hbm.at[pl.ds(src, CHW)],
                             chunk_v.at[pl.ds(slot * CHW, CHW)], sem)

    @pl.when((i & 1) == 0)
    def _():
        _wait(sem0, 0)
        _issue(sem1, 1)

    @pl.when((i & 1) == 1)
    def _():
        _wait(sem1, 1)
        _issue(sem0, 0)


@functools.partial(
    pl.kernel, mesh=_SC_MESH, compiler_params=_SC_PARAMS,
    out_type=jax.ShapeDtypeStruct((NT * 128,), jnp.int32),
    scratch_types=[pltpu.VMEM((128,), jnp.int32),
                   pltpu.VMEM((2 * CHW,), jnp.int32),
                   pltpu.SemaphoreType.DMA, pltpu.SemaphoreType.DMA],
)
def _sc_count(idx_hbm, out_hbm, cnt_v, chunk_v, sem0, sem1):
    tid = lax.axis_index("sc_tile") * 2 + lax.axis_index("sc_core")
    base = tid * FPT
    for k in range(8):
        cnt_v[pl.ds(k * 16, 16)] = jnp.zeros((16,), jnp.int32)

    _, src0 = _chunk_src(0, base)
    pltpu.async_copy(idx_hbm.at[pl.ds(src0, CHW)],
                     chunk_v.at[pl.ds(0, CHW)], sem0)

    def chunk_body(i, _):
        _dbuf_wait_issue(i, NCH, idx_hbm, base, chunk_v, sem0, sem1)
        rs, _ = _chunk_src(i, base)
        off = (i & 1) * CHW
        lo = i * CHW
        ones = jnp.ones((16,), jnp.int32)

        @pl.when(i < NCH - 1)
        def _():
            def vec_body(j, _):
                v = chunk_v[pl.ds(off + j * 16, 16)]
                s = lax.shift_right_logical(v, SLAB_BITS)
                plsc.addupdate_scatter(cnt_v, [s], ones)
                return 0
            lax.fori_loop(0, CHW // 16, vec_body, 0)

        @pl.when(i == NCH - 1)
        def _():
            def vec_body(j, _):
                v = chunk_v[pl.ds(off + j * 16, 16)]
                pos = rs + j * 16 + _LANE()
                m = pos >= lo
                s = lax.shift_right_logical(v, SLAB_BITS)
                plsc.addupdate_scatter(cnt_v, [s], ones, mask=m)
                return 0
            lax.fori_loop(0, CHW // 16, vec_body, 0)
        return 0

    lax.fori_loop(0, NCH, chunk_body, 0)
    pltpu.sync_copy(cnt_v, out_hbm.at[pl.ds(pl.multiple_of(tid * 128, 128), 128)])


@functools.partial(
    pl.kernel, mesh=_SC_MESH, compiler_params=_SC_PARAMS,
    out_type=jax.ShapeDtypeStruct((PART,), jnp.int32),
    scratch_types=[pltpu.VMEM((128,), jnp.int32),
                   pltpu.VMEM((128,), jnp.int32),
                   pltpu.VMEM((128,), jnp.int32),
                   pltpu.VMEM((NSLAB * RING,), jnp.int32),
                   pltpu.VMEM((2 * CHW,), jnp.int32),
                   pltpu.SemaphoreType.DMA, pltpu.SemaphoreType.DMA],
)
def _sc_partition(idx_hbm, base_hbm, part_hbm, hbase_v, fill_v, flq_v,
                  rings_v, chunk_v, sem0, sem1):
    tid = lax.axis_index("sc_tile") * 2 + lax.axis_index("sc_core")
    base = tid * FPT
    pltpu.sync_copy(base_hbm.at[pl.ds(pl.multiple_of(tid * 128, 128), 128)],
                    hbase_v)
    for k in range(8):
        fill_v[pl.ds(k * 16, 16)] = jnp.zeros((16,), jnp.int32)
        flq_v[pl.ds(k * 16, 16)] = jnp.zeros((16,), jnp.int32)

    def flush_block(blk):
        def pending_count():
            fi = fill_v[pl.ds(blk * 16, 16)]
            qi = flq_v[pl.ds(blk * 16, 16)]
            return jnp.sum(jnp.where(fi - qi >= QUANT, 1, 0))

        def cond(n):
            return n > 0

        def body(n):
            fi = fill_v[pl.ds(blk * 16, 16)]
            qi = flq_v[pl.ds(blk * 16, 16)]
            m = fi - qi >= QUANT
            lane = jnp.max(plsc.all_reduce_ffs(m))
            s = blk * 16 + lane
            q = _extract(qi, lane)
            b = _extract(hbase_v[pl.ds(blk * 16, 16)], lane)
            ringoff = pl.multiple_of(s * RING + (q & (RING - 1)), QUANT)
            pltpu.sync_copy(rings_v.at[pl.ds(ringoff, QUANT)],
                            part_hbm.at[pl.ds(pl.multiple_of(b + q, QUANT),
                                              QUANT)])
            flq_v[pl.ds(blk * 16, 16)] = qi + jnp.where(_LANE() == lane, QUANT, 0)
            return n - 1

        lax.while_loop(cond, body, pending_count())

    _, src0 = _chunk_src(0, base)
    pltpu.async_copy(idx_hbm.at[pl.ds(src0, CHW)],
                     chunk_v.at[pl.ds(0, CHW)], sem0)

    def chunk_body(i, _):
        _dbuf_wait_issue(i, NCH, idx_hbm, base, chunk_v, sem0, sem1)
        rs, _ = _chunk_src(i, base)
        off = (i & 1) * CHW
        lo = i * CHW

        def append(j, masked):
            v = chunk_v[pl.ds(off + j * 16, 16)]
            if masked:
                pos = rs + j * 16 + _LANE()
                m = pos >= lo
            else:
                m = None
            s = lax.shift_right_logical(v, SLAB_BITS)
            r, lastm = plsc.scan_count(s, mask=m)
            f = plsc.load_gather(fill_v, [s])
            slot = (f + r - 1) & (RING - 1)
            plsc.store_scatter(rings_v, [s * RING + slot], v, mask=m)
            plsc.addupdate_scatter(fill_v, [s], r, mask=lastm)

        @pl.when(i < NCH - 1)
        def _():
            def group_body(g, _):
                for jj in range(16):
                    append(g * 16 + jj, masked=False)
                for blk in range(8):
                    flush_block(blk)
                return 0
            lax.fori_loop(0, CHW // 256, group_body, 0)

        @pl.when(i == NCH - 1)
        def _():
            def group_body(g, _):
                for jj in range(16):
                    append(g * 16 + jj, masked=True)
                for blk in range(8):
                    flush_block(blk)
                return 0
            lax.fori_loop(0, CHW // 256, group_body, 0)
        return 0

    lax.fori_loop(0, NCH, chunk_body, 0)

    def drain_body(s, _):
        f = _gather_scalar(fill_v, s)
        q = _gather_scalar(flq_v, s)
        pend = f - q

        @pl.when(pend > 0)
        def _():
            end = q + QUANT
            for it in range(QUANT // 16):
                p = f + it * 16 + _LANE()
                m = p < end
                slot = p & (RING - 1)
                plsc.store_scatter(rings_v, [s * RING + slot],
                                   jnp.full((16,), SENT, jnp.int32), mask=m)
            b = _gather_scalar(hbase_v, s)
            ringoff = pl.multiple_of(s * RING + (q & (RING - 1)), QUANT)
            pltpu.sync_copy(rings_v.at[pl.ds(ringoff, QUANT)],
                            part_hbm.at[pl.ds(pl.multiple_of(b + q, QUANT),
                                              QUANT)])
        return 0

    lax.fori_loop(0, NSLAB, drain_body, 0)


@functools.partial(
    pl.kernel, mesh=_SC_MESH, compiler_params=_SC_PARAMS,
    out_type=jax.ShapeDtypeStruct((BINS,), jnp.int32),
    scratch_types=[pltpu.VMEM((256,), jnp.int32),
                   pltpu.VMEM((WIN,), jnp.int32),
                   pltpu.VMEM((2 * P2CHW,), jnp.int32),
                   pltpu.SemaphoreType.DMA, pltpu.SemaphoreType.DMA],
)
def _sc_hist(part_hbm, bounds_hbm, hist_hbm, bounds_v, hist_v, chunk_v,
             sem0, sem1):
    tid = lax.axis_index("sc_tile") * 2 + lax.axis_index("sc_core")
    pltpu.sync_copy(bounds_hbm, bounds_v)

    for i in range(8):
        w = tid + i * NT

        @pl.when(w < NWIN)
        def _():
            s = lax.shift_right_logical(w, 1)
            sstart = _gather_scalar(bounds_v, s)
            send = _gather_scalar(bounds_v, 128 + s)
            wbase = w * WIN

            def zero_body(z, _):
                for k in range(8):
                    hist_v[pl.ds((z * 8 + k) * 16, 16)] = jnp.zeros((16,), jnp.int32)
                return 0

            lax.fori_loop(0, WIN // 128, zero_body, 0)

            n = send - sstart
            trips = lax.shift_right_logical(n + P2CHW - 1, P2BITS)

            def trip_rs(t):
                return pl.multiple_of(
                    jnp.minimum(sstart + t * P2CHW,
                                jnp.maximum(send - P2CHW, 0)), QUANT)

            @pl.when(trips > 0)
            def _():
                pltpu.async_copy(part_hbm.at[pl.ds(trip_rs(0), P2CHW)],
                                 chunk_v.at[pl.ds(0, P2CHW)], sem0)

            def trip_body(t, _):
                def _wait(sem, slot):
                    pltpu.make_async_copy(
                        part_hbm.at[pl.ds(trip_rs(t), P2CHW)],
                        chunk_v.at[pl.ds(slot * P2CHW, P2CHW)], sem).wait()

                def _issue(sem, slot):
                    @pl.when(t + 1 < trips)
                    def _():
                        pltpu.async_copy(
                            part_hbm.at[pl.ds(trip_rs(t + 1), P2CHW)],
                            chunk_v.at[pl.ds(slot * P2CHW, P2CHW)], sem)

                @pl.when((t & 1) == 0)
                def _():
                    _wait(sem0, 0)
                    _issue(sem1, 1)

                @pl.when((t & 1) == 1)
                def _():
                    _wait(sem1, 1)
                    _issue(sem0, 0)

                lo = sstart + t * P2CHW
                hi = jnp.minimum(lo + P2CHW, send)
                rs = trip_rs(t)
                off = (t & 1) * P2CHW
                ones = jnp.ones((16,), jnp.int32)

                def vec_body(j, _):
                    v = chunk_v[pl.ds(off + j * 16, 16)]
                    pos = rs + j * 16 + _LANE()
                    lb = v - wbase
                    m = (pos >= lo) & (pos < hi) & (lb >= 0) & (lb < WIN)
                    lbs = jnp.where(m, lb, 0)
                    plsc.addupdate_scatter(hist_v, [lbs], ones, mask=m)
                    return 0

                lax.fori_loop(0, P2CHW // 16, vec_body, 0)
                return 0

            lax.fori_loop(0, trips, trip_body, 0)
            pltpu.sync_copy(hist_v,
                            hist_hbm.at[pl.ds(pl.multiple_of(w * WIN, WIN),
                                              WIN)])


def _ln_factorial(n_f32):
    x = jnp.maximum(n_f32, 2.0)
    inv = 1.0 / x
    inv2 = inv * inv
    series = inv * (1.0 / 12.0 + inv2 * (-1.0 / 360.0 + inv2 * (1.0 / 1260.0)))
    half_ln_2pi = 0.9189385332046727
    stir = (x + 0.5) * jnp.log(x) - x + half_ln_2pi + series
    return jnp.where(n_f32 < 1.5, 0.0, stir)


def _etable_body(b_ref, h_ref, a_ref, m_ref, e_ref):
    u = (a_ref[...][:, :, None] * b_ref[...][None, :, :]
         + h_ref[...][None, :, :] + m_ref[...][:, :, None])
    e_ref[...] = jnp.sum(jnp.exp(u), axis=-1)


def _reduce_body(cnt_ref, b_ref, h_ref, a_ref, m_ref, e_ref, out_ref):
    cnt = cnt_ref[...].astype(jnp.float32)
    u = (a_ref[...][:, :, None] * b_ref[...][None, :, :]
         + h_ref[...][None, :, :] + m_ref[...][:, :, None])
    t = cnt * u - _ln_factorial(cnt)
    out_ref[...] = jnp.sum(t, axis=-1) - e_ref[...]


def kernel(bincounts, genes_oi, labels, local_cellxgene_ix, binixs,
           baseline_weight, differential_weight, cluster_modifier):
    b = bincounts.astype(jnp.float32)
    h = jnp.take(baseline_weight, genes_oi, axis=0)
    a_k = differential_weight.reshape(K, 1)
    m_k = cluster_modifier.reshape(K, 1)

    e_tab = pl.pallas_call(
        _etable_body,
        out_shape=jax.ShapeDtypeStruct((K, G), jnp.float32),
    )(b, h, a_k, m_k)

    a_c = jnp.take(a_k[:, 0], labels)[:, None]
    m_c = jnp.take(m_k[:, 0], labels)[:, None]
    e_c = jnp.take(e_tab, labels, axis=0)

    idx = _pack(local_cellxgene_ix, binixs)
    counts = _sc_count(idx)

    cnt = counts.reshape(NT, 128)[:, :NSLAB]
    q = ((cnt + (QUANT - 1)) // QUANT) * QUANT
    flat = q.T.reshape(-1)
    starts = jnp.cumsum(flat) - flat
    base_ts = starts.reshape(NSLAB, NT).T
    base_in = jnp.zeros((NT, 128), jnp.int32).at[:, :NSLAB].set(base_ts)
    slab_tot = jnp.sum(q, axis=0)
    slab_start = jnp.cumsum(slab_tot) - slab_tot
    bounds = jnp.zeros((256,), jnp.int32)
    bounds = bounds.at[:NSLAB].set(slab_start)
    bounds = bounds.at[128:128 + NSLAB].set(slab_start + slab_tot)

    part = _sc_partition(idx, base_in.reshape(-1))
    hist = _sc_hist(part, bounds)

    out = pl.pallas_call(
        _reduce_body,
        grid=(C // CB,),
        in_specs=[
            pl.BlockSpec((CB, G, F), lambda i: (i, 0, 0)),
            pl.BlockSpec((G, F), lambda i: (0, 0)),
            pl.BlockSpec((G, F), lambda i: (0, 0)),
            pl.BlockSpec((CB, 1), lambda i: (i, 0)),
            pl.BlockSpec((CB, 1), lambda i: (i, 0)),
            pl.BlockSpec((CB, G), lambda i: (i, 0)),
        ],
        out_specs=pl.BlockSpec((CB, G), lambda i: (i, 0)),
        out_shape=jax.ShapeDtypeStruct((C, G), jnp.float32),
    )(hist.reshape(C, G, F), b, h, a_c, m_c, e_c)
    return out


# speedup vs baseline: 14.9090x; 1.0495x over previous
import functools

import jax
import jax.numpy as jnp
from jax import lax
from jax.experimental import pallas as pl
from jax.experimental.pallas import tpu as pltpu
from jax.experimental.pallas import tpu_sc as plsc

C = 512
G = 100
F = 320
K = 16
CB = 8
NFRAG = 8_000_000
BINS = C * G * F

NT = 32
FPT = NFRAG // NT
SLAB_BITS = 17
NSLAB = BINS >> SLAB_BITS
WIN = 1 << 16
NWIN = BINS // WIN
QUANT = 256
RING = 512
PART = NFRAG + NT * NSLAB * QUANT
SENT = 1 << 30

CHW = 16384
NCH = -(-FPT // CHW)
P2CHW = 8192
P2BITS = 13

_SC_MESH = plsc.VectorSubcoreMesh(core_axis_name="sc_core",
                                  subcore_axis_name="sc_tile")
_SC_PARAMS = pltpu.CompilerParams(needs_layout_passes=False)

_LANE = lambda: lax.broadcasted_iota(jnp.int32, (16,), 0)


def _extract(vec16, lane):
    return jnp.sum(jnp.where(_LANE() == lane, vec16, 0))


def _gather_scalar(ref, idx_scalar):
    g = plsc.load_gather(ref, [jnp.full((16,), idx_scalar, jnp.int32)])
    return jnp.max(g)


def _pack_body(cxg_ref, bin_ref, out_ref):
    out_ref[...] = cxg_ref[...] * F + bin_ref[...]


def _pack(cxg, bins):
    rows, cols = 1000, 8000
    out = pl.pallas_call(
        _pack_body,
        grid=(125,),
        in_specs=[pl.BlockSpec((8, cols), lambda i: (i, 0)),
                  pl.BlockSpec((8, cols), lambda i: (i, 0))],
        out_specs=pl.BlockSpec((8, cols), lambda i: (i, 0)),
        out_shape=jax.ShapeDtypeStruct((rows, cols), jnp.int32),
    )(cxg.reshape(rows, cols), bins.reshape(rows, cols))
    return out.reshape(NFRAG)


def _chunk_src(i, base):
    rs = jnp.minimum(i * CHW, FPT - CHW)
    return rs, pl.multiple_of(base + rs, 16)


def _dbuf_wait_issue(i, nch, idx_hbm, base, chunk_v, sem0, sem1):
    def _wait(sem, slot):
        _, src = _chunk_src(i, base)
        pltpu.make_async_copy(idx_hbm.at[pl.ds(src, CHW)],
                              chunk_v.at[pl.ds(slot * CHW, CHW)], sem).wait()

    def _issue(sem, slot):
        @pl.when(i + 1 < nch)
        def _():
            _, src = _chunk_src(i + 1, base)
            pltpu.async_copy(idx_hbm.at[pl.ds(src, CHW)],
                             chunk_v.at[pl.ds(slot * CHW, CHW)], sem)

    @pl.when((i & 1) == 0)
    def _():
        _wait(sem0, 0)
        _issue(sem1, 1)

    @pl.when((i & 1) == 1)
    def _():
        _wait(sem1, 1)
        _issue(sem0, 0)


@functools.partial(
    pl.kernel, mesh=_SC_MESH, compiler_params=_SC_PARAMS,
    out_type=jax.ShapeDtypeStruct((NT * 128,), jnp.int32),
    scratch_types=[pltpu.VMEM((128,), jnp.int32),
                   pltpu.VMEM((2 * CHW,), jnp.int32),
                   pltpu.SemaphoreType.DMA, pltpu.SemaphoreType.DMA],
)
def _sc_count(idx_hbm, out_hbm, cnt_v, chunk_v, sem0, sem1):
    tid = lax.axis_index("sc_tile") * 2 + lax.axis_index("sc_core")
    base = tid * FPT
    for k in range(8):
        cnt_v[pl.ds(k * 16, 16)] = jnp.zeros((16,), jnp.int32)

    _, src0 = _chunk_src(0, base)
    pltpu.async_copy(idx_hbm.at[pl.ds(src0, CHW)],
                     chunk_v.at[pl.ds(0, CHW)], sem0)

    def chunk_body(i, _):
        _dbuf_wait_issue(i, NCH, idx_hbm, base, chunk_v, sem0, sem1)
        rs, _ = _chunk_src(i, base)
        off = (i & 1) * CHW
        lo = i * CHW
        ones = jnp.ones((16,), jnp.int32)

        @pl.when(i < NCH - 1)
        def _():
            def vec_body(j, _):
                v = chunk_v[pl.ds(off + j * 16, 16)]
                s = lax.shift_right_logical(v, SLAB_BITS)
                plsc.addupdate_scatter(cnt_v, [s], ones)
                return 0
            lax.fori_loop(0, CHW // 16, vec_body, 0)

        @pl.when(i == NCH - 1)
        def _():
            def vec_body(j, _):
                v = chunk_v[pl.ds(off + j * 16, 16)]
                pos = rs + j * 16 + _LANE()
                m = pos >= lo
                s = lax.shift_right_logical(v, SLAB_BITS)
                plsc.addupdate_scatter(cnt_v, [s], ones, mask=m)
                return 0
            lax.fori_loop(0, CHW // 16, vec_body, 0)
        return 0

    lax.fori_loop(0, NCH, chunk_body, 0)
    pltpu.sync_copy(cnt_v, out_hbm.at[pl.ds(pl.multiple_of(tid * 128, 128), 128)])


@functools.partial(
    pl.kernel, mesh=_SC_MESH, compiler_params=_SC_PARAMS,
    out_type=jax.ShapeDtypeStruct((PART,), jnp.int32),
    scratch_types=[pltpu.VMEM((128,), jnp.int32),
                   pltpu.VMEM((128,), jnp.int32),
                   pltpu.VMEM((128,), jnp.int32),
                   pltpu.VMEM((NSLAB * RING,), jnp.int32),
                   pltpu.VMEM((2 * CHW,), jnp.int32),
                   pltpu.SemaphoreType.DMA, pltpu.SemaphoreType.DMA],
)
def _sc_partition(idx_hbm, base_hbm, part_hbm, hbase_v, fill_v, flq_v,
                  rings_v, chunk_v, sem0, sem1):
    tid = lax.axis_index("sc_tile") * 2 + lax.axis_index("sc_core")
    base = tid * FPT
    pltpu.sync_copy(base_hbm.at[pl.ds(pl.multiple_of(tid * 128, 128), 128)],
                    hbase_v)
    for k in range(8):
        fill_v[pl.ds(k * 16, 16)] = jnp.zeros((16,), jnp.int32)
        flq_v[pl.ds(k * 16, 16)] = jnp.zeros((16,), jnp.int32)

    def flush_block(blk):
        def pending_count():
            fi = fill_v[pl.ds(blk * 16, 16)]
            qi = flq_v[pl.ds(blk * 16, 16)]
            return jnp.sum(jnp.where(fi - qi >= QUANT, 1, 0))

        def cond(n):
            return n > 0

        def body(n):
            fi = fill_v[pl.ds(blk * 16, 16)]
            qi = flq_v[pl.ds(blk * 16, 16)]
            m = fi - qi >= QUANT
            lane = jnp.max(plsc.all_reduce_ffs(m))
            s = blk * 16 + lane
            q = _extract(qi, lane)
            b = _extract(hbase_v[pl.ds(blk * 16, 16)], lane)
            ringoff = pl.multiple_of(s * RING + (q & (RING - 1)), QUANT)
            pltpu.sync_copy(rings_v.at[pl.ds(ringoff, QUANT)],
                            part_hbm.at[pl.ds(pl.multiple_of(b + q, QUANT),
                                              QUANT)])
            flq_v[pl.ds(blk * 16, 16)] = qi + jnp.where(_LANE() == lane, QUANT, 0)
            return n - 1

        lax.while_loop(cond, body, pending_count())

    _, src0 = _chunk_src(0, base)
    pltpu.async_copy(idx_hbm.at[pl.ds(src0, CHW)],
                     chunk_v.at[pl.ds(0, CHW)], sem0)

    def chunk_body(i, _):
        _dbuf_wait_issue(i, NCH, idx_hbm, base, chunk_v, sem0, sem1)
        rs, _ = _chunk_src(i, base)
        off = (i & 1) * CHW
        lo = i * CHW

        def append(j, masked):
            v = chunk_v[pl.ds(off + j * 16, 16)]
            if masked:
                pos = rs + j * 16 + _LANE()
                m = pos >= lo
            else:
                m = None
            s = lax.shift_right_logical(v, SLAB_BITS)
            r, lastm = plsc.scan_count(s, mask=m)
            f = plsc.load_gather(fill_v, [s])
            slot = (f + r - 1) & (RING - 1)
            plsc.store_scatter(rings_v, [s * RING + slot], v, mask=m)
            plsc.addupdate_scatter(fill_v, [s], r, mask=lastm)

        def flush_all():
            acc = jnp.zeros((16,), jnp.int32)
            for blk in range(8):
                fi = fill_v[pl.ds(blk * 16, 16)]
                qi = flq_v[pl.ds(blk * 16, 16)]
                acc = acc + jnp.where(fi - qi >= QUANT, 1, 0)
            any_pending = jnp.sum(acc)

            @pl.when(any_pending > 0)
            def _():
                for blk in range(8):
                    flush_block(blk)

        @pl.when(i < NCH - 1)
        def _():
            def group_body(g, _):
                for jj in range(16):
                    append(g * 16 + jj, masked=False)
                flush_all()
                return 0
            lax.fori_loop(0, CHW // 256, group_body, 0)

        @pl.when(i == NCH - 1)
        def _():
            def group_body(g, _):
                for jj in range(16):
                    append(g * 16 + jj, masked=True)
                flush_all()
                return 0
            lax.fori_loop(0, CHW // 256, group_body, 0)
        return 0

    lax.fori_loop(0, NCH, chunk_body, 0)

    def drain_body(s, _):
        f = _gather_scalar(fill_v, s)
        q = _gather_scalar(flq_v, s)
        pend = f - q

        @pl.when(pend > 0)
        def _():
            end = q + QUANT
            for it in range(QUANT // 16):
                p = f + it * 16 + _LANE()
                m = p < end
                slot = p & (RING - 1)
                plsc.store_scatter(rings_v, [s * RING + slot],
                                   jnp.full((16,), SENT, jnp.int32), mask=m)
            b = _gather_scalar(hbase_v, s)
            ringoff = pl.multiple_of(s * RING + (q & (RING - 1)), QUANT)
            pltpu.sync_copy(rings_v.at[pl.ds(ringoff, QUANT)],
                            part_hbm.at[pl.ds(pl.multiple_of(b + q, QUANT),
                                              QUANT)])
        return 0

    lax.fori_loop(0, NSLAB, drain_body, 0)


@functools.partial(
    pl.kernel, mesh=_SC_MESH, compiler_params=_SC_PARAMS,
    out_type=jax.ShapeDtypeStruct((BINS,), jnp.int32),
    scratch_types=[pltpu.VMEM((256,), jnp.int32),
                   pltpu.VMEM((WIN,), jnp.int32),
                   pltpu.VMEM((2 * P2CHW,), jnp.int32),
                   pltpu.SemaphoreType.DMA, pltpu.SemaphoreType.DMA],
)
def _sc_hist(part_hbm, bounds_hbm, hist_hbm, bounds_v, hist_v, chunk_v,
             sem0, sem1):
    tid = lax.axis_index("sc_tile") * 2 + lax.axis_index("sc_core")
    pltpu.sync_copy(bounds_hbm, bounds_v)

    for i in range(8):
        w = tid + i * NT

        @pl.when(w < NWIN)
        def _():
            s = lax.shift_right_logical(w, 1)
            sstart = _gather_scalar(bounds_v, s)
            send = _gather_scalar(bounds_v, 128 + s)
            wbase = w * WIN

            def zero_body(z, _):
                for k in range(8):
                    hist_v[pl.ds((z * 8 + k) * 16, 16)] = jnp.zeros((16,), jnp.int32)
                return 0

            lax.fori_loop(0, WIN // 128, zero_body, 0)

            n = send - sstart
            trips = lax.shift_right_logical(n + P2CHW - 1, P2BITS)

            def trip_rs(t):
                return pl.multiple_of(
                    jnp.minimum(sstart + t * P2CHW,
                                jnp.maximum(send - P2CHW, 0)), QUANT)

            @pl.when(trips > 0)
            def _():
                pltpu.async_copy(part_hbm.at[pl.ds(trip_rs(0), P2CHW)],
                                 chunk_v.at[pl.ds(0, P2CHW)], sem0)

            def trip_body(t, _):
                def _wait(sem, slot):
                    pltpu.make_async_copy(
                        part_hbm.at[pl.ds(trip_rs(t), P2CHW)],
                        chunk_v.at[pl.ds(slot * P2CHW, P2CHW)], sem).wait()

                def _issue(sem, slot):
                    @pl.when(t + 1 < trips)
                    def _():
                        pltpu.async_copy(
                            part_hbm.at[pl.ds(trip_rs(t + 1), P2CHW)],
                            chunk_v.at[pl.ds(slot * P2CHW, P2CHW)], sem)

                @pl.when((t & 1) == 0)
                def _():
                    _wait(sem0, 0)
                    _issue(sem1, 1)

                @pl.when((t & 1) == 1)
                def _():
                    _wait(sem1, 1)
                    _issue(sem0, 0)

                lo = sstart + t * P2CHW
                hi = jnp.minimum(lo + P2CHW, send)
                rs = trip_rs(t)
                off = (t & 1) * P2CHW
                ones = jnp.ones((16,), jnp.int32)

                def vec_body(j, _):
                    v = chunk_v[pl.ds(off + j * 16, 16)]
                    pos = rs + j * 16 + _LANE()
                    lb = v - wbase
                    m = (pos >= lo) & (pos < hi) & (lb >= 0) & (lb < WIN)
                    lbs = jnp.where(m, lb, 0)
                    plsc.addupdate_scatter(hist_v, [lbs], ones, mask=m)
                    return 0

                lax.fori_loop(0, P2CHW // 16, vec_body, 0)
                return 0

            lax.fori_loop(0, trips, trip_body, 0)
            pltpu.sync_copy(hist_v,
                            hist_hbm.at[pl.ds(pl.multiple_of(w * WIN, WIN),
                                              WIN)])


def _ln_factorial(n_f32):
    x = jnp.maximum(n_f32, 2.0)
    inv = 1.0 / x
    inv2 = inv * inv
    series = inv * (1.0 / 12.0 + inv2 * (-1.0 / 360.0 + inv2 * (1.0 / 1260.0)))
    half_ln_2pi = 0.9189385332046727
    stir = (x + 0.5) * jnp.log(x) - x + half_ln_2pi + series
    return jnp.where(n_f32 < 1.5, 0.0, stir)


def _etable_body(b_ref, h_ref, a_ref, m_ref, e_ref):
    u = (a_ref[...][:, :, None] * b_ref[...][None, :, :]
         + h_ref[...][None, :, :] + m_ref[...][:, :, None])
    e_ref[...] = jnp.sum(jnp.exp(u), axis=-1)


def _reduce_body(cnt_ref, b_ref, h_ref, a_ref, m_ref, e_ref, out_ref):
    cnt = cnt_ref[...].astype(jnp.float32)
    u = (a_ref[...][:, :, None] * b_ref[...][None, :, :]
         + h_ref[...][None, :, :] + m_ref[...][:, :, None])
    t = cnt * u - _ln_factorial(cnt)
    out_ref[...] = jnp.sum(t, axis=-1) - e_ref[...]


def kernel(bincounts, genes_oi, labels, local_cellxgene_ix, binixs,
           baseline_weight, differential_weight, cluster_modifier):
    b = bincounts.astype(jnp.float32)
    h = jnp.take(baseline_weight, genes_oi, axis=0)
    a_k = differential_weight.reshape(K, 1)
    m_k = cluster_modifier.reshape(K, 1)

    e_tab = pl.pallas_call(
        _etable_body,
        out_shape=jax.ShapeDtypeStruct((K, G), jnp.float32),
    )(b, h, a_k, m_k)

    a_c = jnp.take(a_k[:, 0], labels)[:, None]
    m_c = jnp.take(m_k[:, 0], labels)[:, None]
    e_c = jnp.take(e_tab, labels, axis=0)

    idx = _pack(local_cellxgene_ix, binixs)
    counts = _sc_count(idx)

    cnt = counts.reshape(NT, 128)[:, :NSLAB]
    q = ((cnt + (QUANT - 1)) // QUANT) * QUANT
    flat = q.T.reshape(-1)
    starts = jnp.cumsum(flat) - flat
    base_ts = starts.reshape(NSLAB, NT).T
    base_in = jnp.zeros((NT, 128), jnp.int32).at[:, :NSLAB].set(base_ts)
    slab_tot = jnp.sum(q, axis=0)
    slab_start = jnp.cumsum(slab_tot) - slab_tot
    bounds = jnp.zeros((256,), jnp.int32)
    bounds = bounds.at[:NSLAB].set(slab_start)
    bounds = bounds.at[128:128 + NSLAB].set(slab_start + slab_tot)

    part = _sc_partition(idx, base_in.reshape(-1))
    hist = _sc_hist(part, bounds)

    out = pl.pallas_call(
        _reduce_body,
        grid=(C // CB,),
        in_specs=[
            pl.BlockSpec((CB, G, F), lambda i: (i, 0, 0)),
            pl.BlockSpec((G, F), lambda i: (0, 0)),
            pl.BlockSpec((G, F), lambda i: (0, 0)),
            pl.BlockSpec((CB, 1), lambda i: (i, 0)),
            pl.BlockSpec((CB, 1), lambda i: (i, 0)),
            pl.BlockSpec((CB, G), lambda i: (i, 0)),
        ],
        out_specs=pl.BlockSpec((CB, G), lambda i: (i, 0)),
        out_shape=jax.ShapeDtypeStruct((C, G), jnp.float32),
    )(hist.reshape(C, G, F), b, h, a_c, m_c, e_c)
    return out
